# baseline (speedup 1.0000x reference)
import functools

import jax
import jax.numpy as jnp
from jax import lax
from jax.experimental import pallas as pl
from jax.experimental.pallas import tpu as pltpu
from jax.experimental.pallas import tpu_sc as plsc

NUM_ITEMS = 1000000
HIDDEN = 64
BATCH = 16384
NC, NS = 2, 16
NW = NC * NS
NR = 2
RNG_RAW = NUM_ITEMS // (NW * NR)
RNG = 15632
NCAP = 384
NSEG = NCAP // 128
SPILL = 8
OUTB = BATCH + SPILL


def _gather_sc(idx, pflat, cflat):
    mesh = plsc.VectorSubcoreMesh(core_axis_name="c", subcore_axis_name="s")

    @functools.partial(
        pl.kernel,
        mesh=mesh,
        compiler_params=pltpu.CompilerParams(
            use_tc_tiling_on_sc=False, needs_layout_passes=False),
        out_type=(
            jax.ShapeDtypeStruct((OUTB, HIDDEN), jnp.float32),
            jax.ShapeDtypeStruct((OUTB, HIDDEN), jnp.float32),
        ),
        scratch_types=[
            pltpu.VMEM((1024,), jnp.int32),
            pltpu.VMEM((NCAP,), jnp.int32),
            pltpu.VMEM((NCAP,), jnp.int32),
            pltpu.VMEM((NSEG, 128), jnp.int32),
            pltpu.VMEM((RNG,), jnp.float32),
            pltpu.VMEM((RNG,), jnp.float32),
            pltpu.VMEM((HIDDEN, NCAP), jnp.float32),
            pltpu.VMEM((HIDDEN, NCAP), jnp.float32),
            pltpu.VMEM((NCAP, HIDDEN), jnp.float32),
            pltpu.SemaphoreType.DMA,
            pltpu.SemaphoreType.DMA,
            pltpu.SemaphoreType.DMA,
        ],
    )
    def k(idx_hbm, p_hbm, c_hbm, p_out, c_out,
          scan_v, l_v, b_v, b2_v, pch_v, cch_v, pst_v, cst_v, rows_v,
          sem_i, sem_p, sem_c):
        wid = lax.axis_index("c") * NS + lax.axis_index("s")

        def round_body(r, carry0):
            start_raw = (wid * NR + r) * RNG_RAW
            start = jnp.minimum((start_raw // 8) * 8, NUM_ITEMS - RNG)
            start = pl.multiple_of(start, 8)

            def prefill(k2, cnt):
                s16 = pl.ds(k2 * 16, 16)
                l_v[s16] = jnp.zeros((16,), jnp.int32)
                b_v[s16] = BATCH + (wid % SPILL) + jnp.zeros((16,), jnp.int32)
                return cnt

            lax.fori_loop(0, NCAP // 16, prefill, 0)

            def scan_piece(p2, cnt):
                pltpu.sync_copy(idx_hbm.at[pl.ds(p2 * 1024, 1024)], scan_v)

                def scan_vec(v, cnt2):
                    ivec = scan_v[pl.ds(v * 16, 16)]
                    lvec = ivec - start
                    rel = ivec - start_raw
                    m = (rel >= 0) & (rel < RNG_RAW)
                    bvec = lax.iota(jnp.int32, 16) + (p2 * 1024 + v * 16)
                    plsc.store_compressed(l_v.at[pl.ds(cnt2, 16)], lvec, mask=m)
                    plsc.store_compressed(b_v.at[pl.ds(cnt2, 16)], bvec, mask=m)
                    npop = plsc.all_reduce_population_count(m)[0]
                    return cnt2 + npop

                return lax.fori_loop(0, 64, scan_vec, cnt)

            lax.fori_loop(0, 16, scan_piece, 0)
            for seg in range(NSEG):
                for k2 in range(8):
                    b2_v[seg, pl.ds(k2 * 16, 16)] = (
                        b_v[pl.ds(seg * 128 + k2 * 16, 16)])

            def percol(c, carry):
                src = pl.multiple_of(c * NUM_ITEMS + start, 8)
                wp = pltpu.async_copy(p_hbm.at[pl.ds(src, RNG)], pch_v, sem_p)
                wc = pltpu.async_copy(c_hbm.at[pl.ds(src, RNG)], cch_v, sem_c)
                wp.wait()
                wc.wait()
                for k2 in range(NCAP // 16):
                    s16 = pl.ds(k2 * 16, 16)
                    lvec = l_v[s16]
                    pst_v[c, s16] = plsc.load_gather(pch_v, [lvec])
                    cst_v[c, s16] = plsc.load_gather(cch_v, [lvec])
                return carry

            lax.fori_loop(0, HIDDEN, percol, 0)

            rows16 = [lax.iota(jnp.int32, 16) + 16 * q
                      for q in range(HIDDEN // 16)]

            for (st, out) in ((pst_v, p_out), (cst_v, c_out)):
                def xpose(j, carry):
                    cols = jnp.broadcast_to(j, (16,))
                    for q in range(HIDDEN // 16):
                        rows_v[j, pl.ds(16 * q, 16)] = plsc.load_gather(
                            st, [rows16[q], cols])
                    return carry

                lax.fori_loop(0, NCAP, xpose, 0)
                waits = []
                for seg in range(NSEG):
                    waits.append(pltpu.async_copy(
                        rows_v.at[pl.ds(seg * 128, 128)],
                        out.at[b2_v.at[seg]], sem_i))
                for w in waits:
                    w.wait()
            return carry0

        lax.fori_loop(0, NR, round_body, 0)

    return k(idx, pflat, cflat)


def _rating_tc(pt, ct, W, b):
    blk = 4096

    def body(p_ref, c_ref, w_ref, b_ref, o_ref):
        s = jnp.sum((p_ref[...] + c_ref[...]) * w_ref[...], axis=0, keepdims=True)
        o_ref[...] = jax.nn.sigmoid(s + b_ref[...])

    return pl.pallas_call(
        body,
        grid=(BATCH // blk,),
        in_specs=[
            pl.BlockSpec((HIDDEN, blk), lambda i: (0, i)),
            pl.BlockSpec((HIDDEN, blk), lambda i: (0, i)),
            pl.BlockSpec((HIDDEN, 1), lambda i: (0, 0)),
            pl.BlockSpec((1, 1), lambda i: (0, 0)),
        ],
        out_specs=pl.BlockSpec((1, blk), lambda i: (0, i)),
        out_shape=jax.ShapeDtypeStruct((1, BATCH), jnp.float32),
    )(pt, ct, W.reshape(HIDDEN, 1), b.reshape(1, 1))


def kernel(item_indices, item_personality_table, item_commonality_table, W, b):
    idx = item_indices.astype(jnp.int32)
    p_ext, c_ext = _gather_sc(
        idx,
        item_personality_table.T.reshape(-1),
        item_commonality_table.T.reshape(-1))
    p = p_ext[:BATCH]
    c = c_ext[:BATCH]
    rating = _rating_tc(p.T, c.T, W, b).reshape(BATCH, 1)
    return (rating, p, c)


# speedup vs baseline: 14.5281x; 14.5281x over previous
import functools

import jax
import jax.numpy as jnp
from jax import lax
from jax.experimental import pallas as pl
from jax.experimental.pallas import tpu as pltpu
from jax.experimental.pallas import tpu_sc as plsc

NUM_ITEMS = 1000000
HIDDEN = 64
BATCH = 16384
NC, NS = 2, 16
NW = NC * NS
NR = 4
NWR = NW * NR
TAIL0 = 999936
NTC_E = TAIL0 // 128
CPR = NTC_E // NWR
CREM = NTC_E % NWR
WCOLS = 62
WIN = WCOLS * 128
NCAP = 256
NSEG = NCAP // 128
SPILL = 8
OUTB = BATCH + SPILL
ROW = 128


def _range_bounds(wr):
    col0 = wr * CPR + jnp.minimum(wr, CREM)
    ncols = jnp.where(wr < CREM, CPR + 1, CPR)
    scol = jnp.minimum(col0, (NUM_ITEMS - WIN) // 128)
    return col0 * 128, (col0 + ncols) * 128, scol * 128


def _select_sc(idx):
    mesh = plsc.VectorSubcoreMesh(core_axis_name="c", subcore_axis_name="s")

    @functools.partial(
        pl.kernel,
        mesh=mesh,
        compiler_params=pltpu.CompilerParams(
            use_tc_tiling_on_sc=False, needs_layout_passes=False),
        out_type=(
            jax.ShapeDtypeStruct((NWR * NCAP,), jnp.int32),
            jax.ShapeDtypeStruct((NWR * NCAP,), jnp.int32),
        ),
        scratch_types=[
            pltpu.VMEM((1024,), jnp.int32),
            pltpu.VMEM((NCAP,), jnp.int32),
            pltpu.VMEM((NCAP,), jnp.int32),
            pltpu.SemaphoreType.DMA,
        ],
    )
    def k(idx_hbm, l_out, b_out, scan_v, l_v, b_v, sem):
        wid = lax.axis_index("c") * NS + lax.axis_index("s")
        lanes = lax.iota(jnp.int32, 16)

        def round_body(r, carry0):
            wr = wid * NR + r
            i_lo, i_hi, base = _range_bounds(wr)

            def prefill(k2, cnt):
                s16 = pl.ds(k2 * 16, 16)
                l_v[s16] = jnp.zeros((16,), jnp.int32)
                b_v[s16] = BATCH + (wid % SPILL) + jnp.zeros((16,), jnp.int32)
                return cnt

            lax.fori_loop(0, NCAP // 16, prefill, 0)

            def scan_piece(p2, cnt):
                pltpu.sync_copy(idx_hbm.at[pl.ds(p2 * 1024, 1024)], scan_v)

                def scan_vec(v, cnt2):
                    ivec = scan_v[pl.ds(v * 16, 16)]
                    m = (ivec >= i_lo) & (ivec < i_hi)
                    bvec = lanes + (p2 * 1024 + v * 16)
                    plsc.store_compressed(
                        l_v.at[pl.ds(cnt2, 16)], ivec - base, mask=m)
                    plsc.store_compressed(
                        b_v.at[pl.ds(cnt2, 16)], bvec, mask=m)
                    return cnt2 + plsc.all_reduce_population_count(m)[0]

                return lax.fori_loop(0, 64, scan_vec, cnt)

            lax.fori_loop(0, 16, scan_piece, 0)
            pltpu.sync_copy(l_v, l_out.at[pl.ds(wr * NCAP, NCAP)])
            pltpu.sync_copy(b_v, b_out.at[pl.ds(wr * NCAP, NCAP)])
            return carry0

        lax.fori_loop(0, NR, round_body, 0)

    return k(idx)


def _stream_sc(l_list, b_list, pt, ct):
    mesh = plsc.VectorSubcoreMesh(core_axis_name="c", subcore_axis_name="s")

    @functools.partial(
        pl.kernel,
        mesh=mesh,
        compiler_params=pltpu.CompilerParams(needs_layout_passes=False),
        out_type=(
            jax.ShapeDtypeStruct((OUTB, ROW), jnp.float32),
            jax.ShapeDtypeStruct((OUTB, ROW), jnp.float32),
        ),
        scratch_types=[
            pltpu.VMEM((NCAP,), jnp.int32),
            pltpu.VMEM((NSEG, 128), jnp.int32),
            pltpu.VMEM((8, WIN), jnp.float32),
            pltpu.VMEM((NCAP, ROW), jnp.float32),
            pltpu.SemaphoreType.DMA,
            pltpu.SemaphoreType.DMA,
        ],
    )
    def k(l_hbm, b_hbm, p_hbm, c_hbm, p_out, c_out,
          l_v, b2_v, ch_v, st_v, sem_d, sem_s):
        wid = lax.axis_index("c") * NS + lax.axis_index("s")
        lanes = lax.iota(jnp.int32, 16)
        rows8 = lanes & 7

        def round_body(r, carry0):
            wr = wid * NR + r
            _, _, base = _range_bounds(wr)
            base = pl.multiple_of(base, 128)
            pltpu.sync_copy(l_hbm.at[pl.ds(wr * NCAP, NCAP)], l_v)
            for seg in range(NSEG):
                pltpu.sync_copy(
                    b_hbm.at[pl.ds(wr * NCAP + seg * 128, 128)],
                    b2_v.at[seg])

            for (tab, out) in ((p_hbm, p_out), (c_hbm, c_out)):
                def ablock(a, carry):
                    pltpu.async_copy(
                        tab.at[pl.ds(a * 8, 8), pl.ds(base, WIN)],
                        ch_v, sem_d).wait()

                    def extract(k2, carry2):
                        lvec = l_v[pl.ds(k2 * 16, 16)]
                        for j in range(16):
                            lj = jnp.broadcast_to(lvec[j], (16,))
                            vals = plsc.load_gather(ch_v, [rows8, lj])
                            pos = jnp.broadcast_to(k2 * 16 + j, (16,))
                            plsc.store_scatter(st_v, [pos, a * 8 + lanes], vals)
                        return carry2

                    lax.fori_loop(0, NCAP // 16, extract, 0)
                    return carry

                lax.fori_loop(0, HIDDEN // 8, ablock, 0)
                waits = []
                for seg in range(NSEG):
                    waits.append(pltpu.async_copy(
                        st_v.at[pl.ds(seg * 128, 128)],
                        out.at[b2_v.at[seg]], sem_s))
                for w in waits:
                    w.wait()
            return carry0

        lax.fori_loop(0, NR, round_body, 0)

    return k(l_list, b_list, pt, ct)


def _rating_tc(pt, ct, W, b):
    blk = 4096

    def body(p_ref, c_ref, w_ref, b_ref, o_ref):
        s = jnp.sum((p_ref[...] + c_ref[...]) * w_ref[...], axis=0, keepdims=True)
        o_ref[...] = jax.nn.sigmoid(s + b_ref[...])

    return pl.pallas_call(
        body,
        grid=(BATCH // blk,),
        in_specs=[
            pl.BlockSpec((HIDDEN, blk), lambda i: (0, i)),
            pl.BlockSpec((HIDDEN, blk), lambda i: (0, i)),
            pl.BlockSpec((HIDDEN, 1), lambda i: (0, 0)),
            pl.BlockSpec((1, 1), lambda i: (0, 0)),
        ],
        out_specs=pl.BlockSpec((1, blk), lambda i: (0, i)),
        out_shape=jax.ShapeDtypeStruct((1, BATCH), jnp.float32),
    )(pt, ct, W.reshape(HIDDEN, 1), b.reshape(1, 1))


def kernel(item_indices, item_personality_table, item_commonality_table, W, b):
    idx = item_indices.astype(jnp.int32)
    l_list, b_list = _select_sc(idx)
    p_ext, c_ext = _stream_sc(
        l_list, b_list, item_personality_table.T, item_commonality_table.T)
    tmask = idx >= TAIL0
    tfix = jnp.where(tmask, idx - TAIL0, 0)
    ptail = jnp.take(item_personality_table[TAIL0:], tfix, axis=0)
    ctail = jnp.take(item_commonality_table[TAIL0:], tfix, axis=0)
    p = jnp.where(tmask[:, None], ptail, p_ext[:BATCH, :HIDDEN])
    c = jnp.where(tmask[:, None], ctail, c_ext[:BATCH, :HIDDEN])
    rating = _rating_tc(p.T, c.T, W, b).reshape(BATCH, 1)
    return (rating, p, c)


# speedup vs baseline: 17.6791x; 1.2169x over previous
import functools

import jax
import jax.numpy as jnp
from jax import lax
from jax.experimental import pallas as pl
from jax.experimental.pallas import tpu as pltpu
from jax.experimental.pallas import tpu_sc as plsc

NUM_ITEMS = 1000000
HIDDEN = 64
BATCH = 16384
NC, NS = 2, 16
NW = NC * NS
NR = 4
NWR = NW * NR
TAIL0 = 999936
NTC_E = TAIL0 // 128
CPR = NTC_E // NWR
CREM = NTC_E % NWR
HWIN = 4096
NCAP = 256
NSEG = NCAP // 128
SPILL = 8
OUTB = BATCH + SPILL
ROW = 128


def _range_bounds(wr):
    col0 = wr * CPR + jnp.minimum(wr, CREM)
    ncols = jnp.where(wr < CREM, CPR + 1, CPR)
    scol = jnp.minimum(col0, (TAIL0 - 2 * HWIN) // 128)
    return col0 * 128, (col0 + ncols) * 128, scol * 128


def _select_sc(idx):
    mesh = plsc.VectorSubcoreMesh(core_axis_name="c", subcore_axis_name="s")

    @functools.partial(
        pl.kernel,
        mesh=mesh,
        compiler_params=pltpu.CompilerParams(
            use_tc_tiling_on_sc=False, needs_layout_passes=False),
        out_type=(
            jax.ShapeDtypeStruct((NWR * NCAP,), jnp.int32),
            jax.ShapeDtypeStruct((NWR * NCAP,), jnp.int32),
        ),
        scratch_types=[
            pltpu.VMEM((1024,), jnp.int32),
            pltpu.VMEM((NR, NCAP), jnp.int32),
            pltpu.VMEM((NR, NCAP), jnp.int32),
            pltpu.SemaphoreType.DMA,
        ],
    )
    def k(idx_hbm, l_out, b_out, scan_v, l_v, b_v, sem):
        wid = lax.axis_index("c") * NS + lax.axis_index("s")
        lanes = lax.iota(jnp.int32, 16)

        for r in range(NR):
            def prefill(k2, cnt, r=r):
                s16 = pl.ds(k2 * 16, 16)
                l_v[r, s16] = jnp.zeros((16,), jnp.int32)
                b_v[r, s16] = BATCH + (wid % SPILL) + jnp.zeros((16,), jnp.int32)
                return cnt

            lax.fori_loop(0, NCAP // 16, prefill, 0)

        bounds = [_range_bounds(wid * NR + r) for r in range(NR)]

        def scan_piece(p2, cnts):
            pltpu.sync_copy(idx_hbm.at[pl.ds(p2 * 1024, 1024)], scan_v)

            def scan_vec(v, cnts2):
                ivec = scan_v[pl.ds(v * 16, 16)]
                bvec = lanes + (p2 * 1024 + v * 16)
                out = []
                for r in range(NR):
                    i_lo, i_hi, base = bounds[r]
                    m = (ivec >= i_lo) & (ivec < i_hi)
                    plsc.store_compressed(
                        l_v.at[r].at[pl.ds(cnts2[r], 16)], ivec - base, mask=m)
                    plsc.store_compressed(
                        b_v.at[r].at[pl.ds(cnts2[r], 16)], bvec, mask=m)
                    out.append(
                        cnts2[r] + plsc.all_reduce_population_count(m)[0])
                return tuple(out)

            return lax.fori_loop(0, 64, scan_vec, cnts)

        lax.fori_loop(0, 16, scan_piece, (0,) * NR)
        for r in range(NR):
            wr_off = (wid * NR + r) * NCAP
            pltpu.sync_copy(l_v.at[r], l_out.at[pl.ds(wr_off, NCAP)])
            pltpu.sync_copy(b_v.at[r], b_out.at[pl.ds(wr_off, NCAP)])

    return k(idx)


def _stream_sc(l_list, b_list, pt, ct):
    mesh = plsc.VectorSubcoreMesh(core_axis_name="c", subcore_axis_name="s")

    @functools.partial(
        pl.kernel,
        mesh=mesh,
        compiler_params=pltpu.CompilerParams(needs_layout_passes=False),
        out_type=(
            jax.ShapeDtypeStruct((OUTB, ROW), jnp.float32),
            jax.ShapeDtypeStruct((OUTB, ROW), jnp.float32),
        ),
        scratch_types=[
            pltpu.VMEM((NCAP,), jnp.int32),
            pltpu.VMEM((NSEG, 128), jnp.int32),
            pltpu.VMEM((8, HWIN), jnp.float32),
            pltpu.VMEM((8, HWIN), jnp.float32),
            pltpu.VMEM((NCAP + 8, ROW), jnp.float32),
            pltpu.SemaphoreType.DMA,
            pltpu.SemaphoreType.DMA,
            pltpu.SemaphoreType.DMA,
        ],
    )
    def k(l_hbm, b_hbm, p_hbm, c_hbm, p_out, c_out,
          l_v, b2_v, chA, chB, st_v, semA, semB, sem_s):
        wid = lax.axis_index("c") * NS + lax.axis_index("s")
        lanes = lax.iota(jnp.int32, 16)
        lo8 = lanes < 8
        rows8 = lanes & 7
        pair01 = jnp.where(lo8, 0, 1)

        def round_body(r, carry0):
            wr = wid * NR + r
            _, _, base = _range_bounds(wr)
            base = pl.multiple_of(base, 128)
            hb1 = pl.multiple_of(
                jnp.minimum(base + HWIN, TAIL0 - HWIN), 128)
            pltpu.sync_copy(l_hbm.at[pl.ds(wr * NCAP, NCAP)], l_v)
            for seg in range(NSEG):
                pltpu.sync_copy(
                    b_hbm.at[pl.ds(wr * NCAP + seg * 128, 128)],
                    b2_v.at[seg])

            for (tab, out) in ((p_hbm, p_out), (c_hbm, c_out)):
                steps = [(a, h) for a in range(HIDDEN // 8) for h in range(2)]

                def fire(t):
                    a, h = steps[t]
                    buf, sem = (chA, semA) if t % 2 == 0 else (chB, semB)
                    hb = base if h == 0 else hb1
                    return pltpu.async_copy(
                        tab.at[pl.ds(a * 8, 8), pl.ds(hb, HWIN)], buf, sem)

                pending = fire(0)
                for t, (a, h) in enumerate(steps):
                    nxt = fire(t + 1) if t + 1 < len(steps) else None
                    pending.wait()
                    pending = nxt
                    buf = chA if t % 2 == 0 else chB
                    hb_rel = (base if h == 0 else hb1) - base

                    def extract(k2, carry2, a=a, buf=buf, hb_rel=hb_rel):
                        lvec = l_v[pl.ds(k2 * 16, 16)]
                        for j in range(0, 16, 2):
                            l0 = jnp.broadcast_to(lvec[j], (16,))
                            l1 = jnp.broadcast_to(lvec[j + 1], (16,))
                            lsel = jnp.where(lo8, l0, l1) - hb_rel
                            valid = (lsel >= 0) & (lsel < HWIN)
                            cols = jnp.clip(lsel, 0, HWIN - 1)
                            vals = plsc.load_gather(buf, [rows8, cols])
                            posb = jnp.broadcast_to(k2 * 16 + j, (16,))
                            rowsel = jnp.where(valid, posb + pair01, NCAP)
                            plsc.store_scatter(
                                st_v, [rowsel, a * 8 + rows8], vals)
                        return carry2

                    lax.fori_loop(0, NCAP // 16, extract, 0)

                waits = []
                for seg in range(NSEG):
                    waits.append(pltpu.async_copy(
                        st_v.at[pl.ds(seg * 128, 128)],
                        out.at[b2_v.at[seg]], sem_s))
                for w in waits:
                    w.wait()
            return carry0

        lax.fori_loop(0, NR, round_body, 0)

    return k(l_list, b_list, pt, ct)


def _rating_tc(pt, ct, W, b):
    blk = 4096

    def body(p_ref, c_ref, w_ref, b_ref, o_ref):
        s = jnp.sum((p_ref[...] + c_ref[...]) * w_ref[...], axis=0, keepdims=True)
        o_ref[...] = jax.nn.sigmoid(s + b_ref[...])

    return pl.pallas_call(
        body,
        grid=(BATCH // blk,),
        in_specs=[
            pl.BlockSpec((HIDDEN, blk), lambda i: (0, i)),
            pl.BlockSpec((HIDDEN, blk), lambda i: (0, i)),
            pl.BlockSpec((HIDDEN, 1), lambda i: (0, 0)),
            pl.BlockSpec((1, 1), lambda i: (0, 0)),
        ],
        out_specs=pl.BlockSpec((1, blk), lambda i: (0, i)),
        out_shape=jax.ShapeDtypeStruct((1, BATCH), jnp.float32),
    )(pt, ct, W.reshape(HIDDEN, 1), b.reshape(1, 1))


def kernel(item_indices, item_personality_table, item_commonality_table, W, b):
    idx = item_indices.astype(jnp.int32)
    l_list, b_list = _select_sc(idx)
    p_ext, c_ext = _stream_sc(
        l_list, b_list, item_personality_table.T, item_commonality_table.T)
    tmask = idx >= TAIL0
    tfix = jnp.where(tmask, idx - TAIL0, 0)
    ptail = jnp.take(item_personality_table[TAIL0:], tfix, axis=0)
    ctail = jnp.take(item_commonality_table[TAIL0:], tfix, axis=0)
    p = jnp.where(tmask[:, None], ptail, p_ext[:BATCH, :HIDDEN])
    c = jnp.where(tmask[:, None], ctail, c_ext[:BATCH, :HIDDEN])
    rating = _rating_tc(p.T, c.T, W, b).reshape(BATCH, 1)
    return (rating, p, c)
